# baseline (speedup 1.0000x reference)
import functools

import jax
import jax.numpy as jnp
from jax import lax
from jax.experimental import pallas as pl
from jax.experimental.pallas import tpu as pltpu
from jax.experimental.pallas import tpu_sc as plsc

NC = 2
NS = 16
LANE = 16
CHUNK = 128
DEG_W = 128


def _cdiv(a, b):
    return (a + b - 1) // b


def _dinv_col(deg_ref):
    s = deg_ref[0:1, :] + deg_ref[1:2, :]
    dinv = jnp.where(s > 0, lax.rsqrt(jnp.where(s > 0, s, 1.0)), 0.0)
    return lax.transpose(dinv, (1, 0))


def _tc_mm_body(x_ref, w_ref, o_ref):
    o_ref[...] = lax.dot_general(x_ref[...], w_ref[...],
                                 (((1,), (1,)), ((), ())),
                                 preferred_element_type=jnp.float32)


def _tc_scale_body(t_ref, deg_ref, o_ref):
    o_ref[...] = t_ref[...] * _dinv_col(deg_ref)


def _tc_mid_body(agg_ref, deg_ref, b_ref, w_ref, o_ref):
    dinv = _dinv_col(deg_ref)
    h = (agg_ref[0] + agg_ref[1]) * dinv + b_ref[...]
    h = jnp.maximum(h, 0.0)
    t = lax.dot_general(h, w_ref[...], (((1,), (1,)), ((), ())),
                        preferred_element_type=jnp.float32)
    o_ref[...] = t * dinv


def _tc_last_body(agg_ref, deg_ref, b_ref, o_ref):
    dinv = _dinv_col(deg_ref)
    o_ref[...] = (agg_ref[0] + agg_ref[1]) * dinv + b_ref[...]


def _row_blocks(n):
    return min(2048, _cdiv(n, 128) * 128)


def _fill_vmem(ref, rows, w, val):
    v = jnp.full((16,), val, jnp.float32)

    def row(i, c):
        for l in range(w // 16):
            ref[i, pl.ds(l * 16, 16)] = v
        return c

    lax.fori_loop(0, rows, row, 0)


def _zero_acc_slice(zbuf, acc, base, zrows):
    nfull, rem = zrows // CHUNK, zrows % CHUNK
    for k in range(nfull):
        pltpu.sync_copy(zbuf, acc.at[pl.ds(base + k * CHUNK, CHUNK)])
    if rem:
        pltpu.sync_copy(zbuf.at[pl.ds(0, rem)],
                        acc.at[pl.ds(base + nfull * CHUNK, rem)])


def _make_sc_deg(n_pad, rw):
    mesh = plsc.VectorSubcoreMesh(core_axis_name="c", subcore_axis_name="s")
    cols = n_pad // NS

    def body(dst_hbm, out_hbm, dst_all, hist, red, osum, stage):
        cid = lax.axis_index("c")
        sid = lax.axis_index("s")
        wid = cid * NS + sid

        def z(i, c):
            hist[pl.ds(i * 16, 16)] = jnp.zeros((16,), jnp.float32)
            return c

        lax.fori_loop(0, n_pad // 16, z, 0)
        pltpu.sync_copy(dst_hbm.at[pl.ds(wid * rw, rw)], dst_all)
        ones16 = jnp.ones((16,), jnp.float32)

        def chunk(j, c):
            for l in range(CHUNK // 16):
                idx = dst_all[j, pl.ds(l * 16, 16)]
                plsc.addupdate_scatter(hist, [idx], ones16)
            return c

        lax.fori_loop(0, rw, chunk, 0)
        pltpu.sync_copy(hist, stage.at[sid])
        plsc.subcore_barrier()
        pltpu.sync_copy(stage.at[:, pl.ds(sid * cols, cols)], red)

        def col(i, c):
            s = jnp.zeros((16,), jnp.float32)
            for r in range(NS):
                s = s + red[r, pl.ds(i * 16, 16)]
            osum[pl.ds(i * 16, 16)] = s
            return c

        lax.fori_loop(0, cols // 16, col, 0)
        pltpu.sync_copy(osum, out_hbm.at[cid, pl.ds(sid * cols, cols)])

    return pl.kernel(
        body,
        out_type=jax.ShapeDtypeStruct((NC, n_pad), jnp.float32),
        mesh=mesh,
        compiler_params=pltpu.CompilerParams(needs_layout_passes=False),
        scratch_types=[
            pltpu.VMEM((rw, CHUNK), jnp.int32),
            pltpu.VMEM((n_pad,), jnp.float32),
            pltpu.VMEM((NS, cols), jnp.float32),
            pltpu.VMEM((cols,), jnp.float32),
            pltpu.VMEM_SHARED((NS, n_pad), jnp.float32),
        ],
    )


NBUF = 2
IB = 40


def _make_sc_agg(n_pad, d, rw0, rw1):
    mesh = plsc.VectorSubcoreMesh(core_axis_name="c", subcore_axis_name="s")
    zrows = n_pad // NS
    orows = n_pad // NS

    def body(t_hbm, src_hbm, dst_hbm, out_hbm,
             src_ib, dst_ib, bufs, acc, gsems, ssems):
        cid = lax.axis_index("c")
        sid = lax.axis_index("s")
        _fill_vmem(bufs.at[0], CHUNK, d, 0.0)
        _zero_acc_slice(bufs.at[0], acc, sid * zrows, zrows)
        plsc.subcore_barrier()

        is0 = cid == 0
        base_chunk = jnp.where(is0, sid * rw0, NS * rw0 + sid * rw1)
        nblk = jnp.where(is0, rw0 // IB, rw1 // IB)

        def block(g, c):
            base = base_chunk + g * IB
            pltpu.sync_copy(src_hbm.at[pl.ds(base, IB)], src_ib)
            pltpu.sync_copy(dst_hbm.at[pl.ds(base, IB)], dst_ib)
            pltpu.async_copy(t_hbm.at[src_ib.at[0]], bufs.at[0], gsems.at[0])
            for j in range(IB):
                b = j % NBUF
                nb = (j + 1) % NBUF
                pltpu.make_async_copy(t_hbm.at[src_ib.at[j]], bufs.at[b],
                                      gsems.at[b]).wait()
                pltpu.async_copy(bufs.at[b], acc.at[dst_ib.at[j]],
                                 ssems.at[b], add=True)
                if j + 1 < IB:
                    if j >= 1:
                        pltpu.make_async_copy(bufs.at[nb],
                                              acc.at[dst_ib.at[j - 1]],
                                              ssems.at[nb]).wait()
                    pltpu.async_copy(t_hbm.at[src_ib.at[j + 1]],
                                     bufs.at[nb], gsems.at[nb])
            for j in range(IB - NBUF, IB):
                b = j % NBUF
                pltpu.make_async_copy(bufs.at[b], acc.at[dst_ib.at[j]],
                                      ssems.at[b]).wait()
            return c

        lax.fori_loop(0, nblk, block, 0)
        plsc.subcore_barrier()
        pltpu.sync_copy(acc.at[pl.ds(sid * orows, orows)],
                        out_hbm.at[cid, pl.ds(sid * orows, orows)])

    return pl.kernel(
        body,
        out_type=jax.ShapeDtypeStruct((NC, n_pad, d), jnp.float32),
        mesh=mesh,
        scratch_types=[
            pltpu.VMEM((IB, CHUNK), jnp.int32),
            pltpu.VMEM((IB, CHUNK), jnp.int32),
            pltpu.VMEM((NBUF, CHUNK, d), jnp.float32),
            pltpu.VMEM_SHARED((n_pad, d), jnp.float32),
            pltpu.SemaphoreType.DMA((NBUF,)),
            pltpu.SemaphoreType.DMA((NBUF,)),
        ],
    )


def kernel(x, edge_index, W1, b1, W2, b2):
    n, d = x.shape
    e = edge_index.shape[1]
    d_hid = W1.shape[0]
    d_out = W2.shape[0]

    nw = NC * NS
    rw = _cdiv(_cdiv(_cdiv(e, CHUNK), nw), IB) * IB
    rw0 = rw1 = rw
    e_pad = NS * (rw0 + rw1) * CHUNK
    n_pad = _cdiv(n + 1, 128) * 128

    src = edge_index[0].astype(jnp.int32)
    dst = edge_index[1].astype(jnp.int32)
    pad_src = jnp.arange(e_pad - e, dtype=jnp.int32) % n
    src2d = jnp.concatenate([src, pad_src]).reshape(nw * rw, CHUNK)
    dst2d = jnp.pad(dst, (0, e_pad - e),
                    constant_values=n).reshape(nw * rw, CHUNK)

    n_pad_deg = _cdiv(n + 1, NS * 128) * NS * 128
    degw = _make_sc_deg(n_pad_deg, rw)(dst2d)

    sc_agg = _make_sc_agg(n_pad, d_hid, rw0, rw1)

    blk = _row_blocks(n)
    grid = (_cdiv(n, blk),)
    degw_spec = pl.BlockSpec((NC, blk), lambda i: (0, i))
    row_spec = pl.BlockSpec((blk, d), lambda i: (i, 0))
    w_spec = pl.BlockSpec((d_hid, d), lambda i: (0, 0))
    b_spec = pl.BlockSpec((1, d_hid), lambda i: (0, 0))
    agg_spec = pl.BlockSpec((NC, blk, d_hid), lambda i: (0, i, 0))

    t1_raw = pl.pallas_call(
        _tc_mm_body,
        grid=grid,
        in_specs=[row_spec, w_spec],
        out_specs=pl.BlockSpec((blk, d_hid), lambda i: (i, 0)),
        out_shape=jax.ShapeDtypeStruct((n, d_hid), jnp.float32),
    )(x, W1)
    t1 = pl.pallas_call(
        _tc_scale_body,
        grid=grid,
        in_specs=[pl.BlockSpec((blk, d_hid), lambda i: (i, 0)), degw_spec],
        out_specs=pl.BlockSpec((blk, d_hid), lambda i: (i, 0)),
        out_shape=jax.ShapeDtypeStruct((n, d_hid), jnp.float32),
    )(t1_raw, degw)

    agg1 = sc_agg(t1, src2d, dst2d)

    t2 = pl.pallas_call(
        _tc_mid_body,
        grid=grid,
        in_specs=[agg_spec, degw_spec, b_spec,
                  pl.BlockSpec((d_out, d_hid), lambda i: (0, 0))],
        out_specs=pl.BlockSpec((blk, d_out), lambda i: (i, 0)),
        out_shape=jax.ShapeDtypeStruct((n, d_out), jnp.float32),
    )(agg1, degw, b1.reshape(1, d_hid), W2)

    agg2 = sc_agg(t2, src2d, dst2d)

    out = pl.pallas_call(
        _tc_last_body,
        grid=grid,
        in_specs=[agg_spec, degw_spec, b_spec],
        out_specs=pl.BlockSpec((blk, d_out), lambda i: (i, 0)),
        out_shape=jax.ShapeDtypeStruct((n, d_out), jnp.float32),
    )(agg2, degw, b2.reshape(1, d_out))

    return out


# speedup vs baseline: 1.1421x; 1.1421x over previous
import functools

import jax
import jax.numpy as jnp
from jax import lax
from jax.experimental import pallas as pl
from jax.experimental.pallas import tpu as pltpu
from jax.experimental.pallas import tpu_sc as plsc

NC = 2
NS = 16
LANE = 16
CHUNK = 128
DEG_W = 128


def _cdiv(a, b):
    return (a + b - 1) // b


def _dinv_col(deg_ref):
    s = deg_ref[0:1, :] + deg_ref[1:2, :]
    dinv = jnp.where(s > 0, lax.rsqrt(jnp.where(s > 0, s, 1.0)), 0.0)
    return lax.transpose(dinv, (1, 0))


def _tc_mm_body(x_ref, w_ref, o_ref):
    o_ref[...] = lax.dot_general(x_ref[...], w_ref[...],
                                 (((1,), (1,)), ((), ())),
                                 preferred_element_type=jnp.float32)


def _tc_scale_body(t_ref, deg_ref, o_ref):
    o_ref[...] = t_ref[...] * _dinv_col(deg_ref)


def _tc_mid_body(agg_ref, deg_ref, b_ref, w_ref, o_ref):
    dinv = _dinv_col(deg_ref)
    h = (agg_ref[0] + agg_ref[1]) * dinv + b_ref[...]
    h = jnp.maximum(h, 0.0)
    t = lax.dot_general(h, w_ref[...], (((1,), (1,)), ((), ())),
                        preferred_element_type=jnp.float32)
    o_ref[...] = t * dinv


def _tc_last_body(agg_ref, deg_ref, b_ref, o_ref):
    dinv = _dinv_col(deg_ref)
    o_ref[...] = (agg_ref[0] + agg_ref[1]) * dinv + b_ref[...]


def _row_blocks(n):
    return min(2048, _cdiv(n, 128) * 128)


def _fill_vmem(ref, rows, w, val):
    v = jnp.full((16,), val, jnp.float32)

    def row(i, c):
        for l in range(w // 16):
            ref[i, pl.ds(l * 16, 16)] = v
        return c

    lax.fori_loop(0, rows, row, 0)


def _zero_acc_slice(zbuf, acc, base, zrows):
    nfull, rem = zrows // CHUNK, zrows % CHUNK
    for k in range(nfull):
        pltpu.sync_copy(zbuf, acc.at[pl.ds(base + k * CHUNK, CHUNK)])
    if rem:
        pltpu.sync_copy(zbuf.at[pl.ds(0, rem)],
                        acc.at[pl.ds(base + nfull * CHUNK, rem)])


def _make_sc_deg(n_pad, rw):
    mesh = plsc.VectorSubcoreMesh(core_axis_name="c", subcore_axis_name="s")
    cols = n_pad // NS

    def body(dst_hbm, out_hbm, dst_all, hist, red, osum, stage):
        cid = lax.axis_index("c")
        sid = lax.axis_index("s")
        wid = cid * NS + sid

        def z(i, c):
            hist[pl.ds(i * 16, 16)] = jnp.zeros((16,), jnp.float32)
            return c

        lax.fori_loop(0, n_pad // 16, z, 0)
        pltpu.sync_copy(dst_hbm.at[pl.ds(wid * rw, rw)], dst_all)
        ones16 = jnp.ones((16,), jnp.float32)

        def chunk(j, c):
            for l in range(CHUNK // 16):
                idx = dst_all[j, pl.ds(l * 16, 16)]
                plsc.addupdate_scatter(hist, [idx], ones16)
            return c

        lax.fori_loop(0, rw, chunk, 0)
        pltpu.sync_copy(hist, stage.at[sid])
        plsc.subcore_barrier()
        pltpu.sync_copy(stage.at[:, pl.ds(sid * cols, cols)], red)

        def col(i, c):
            s = jnp.zeros((16,), jnp.float32)
            for r in range(NS):
                s = s + red[r, pl.ds(i * 16, 16)]
            osum[pl.ds(i * 16, 16)] = s
            return c

        lax.fori_loop(0, cols // 16, col, 0)
        pltpu.sync_copy(osum, out_hbm.at[cid, pl.ds(sid * cols, cols)])

    return pl.kernel(
        body,
        out_type=jax.ShapeDtypeStruct((NC, n_pad), jnp.float32),
        mesh=mesh,
        compiler_params=pltpu.CompilerParams(needs_layout_passes=False),
        scratch_types=[
            pltpu.VMEM((rw, CHUNK), jnp.int32),
            pltpu.VMEM((n_pad,), jnp.float32),
            pltpu.VMEM((NS, cols), jnp.float32),
            pltpu.VMEM((cols,), jnp.float32),
            pltpu.VMEM_SHARED((NS, n_pad), jnp.float32),
        ],
    )


NBUF = 2
IB = 40


def _make_sc_agg(n_pad, d, rw0, rw1):
    mesh = plsc.VectorSubcoreMesh(core_axis_name="c", subcore_axis_name="s")
    zrows = n_pad // NS
    orows = n_pad // NS

    def body(t_hbm, src_hbm, dst_hbm, out_hbm,
             src_ib, dst_ib, bufs, acc, gsems, ssems):
        cid = lax.axis_index("c")
        sid = lax.axis_index("s")
        _fill_vmem(bufs.at[0], CHUNK, d, 0.0)
        _zero_acc_slice(bufs.at[0], acc, sid * zrows, zrows)
        plsc.subcore_barrier()

        is0 = cid == 0
        base_chunk = jnp.where(is0, sid * rw0, NS * rw0 + sid * rw1)
        nblk = jnp.where(is0, rw0 // IB, rw1 // IB)

        def block(g, c):
            base = base_chunk + g * IB
            pltpu.sync_copy(src_hbm.at[pl.ds(base, IB)], src_ib)
            pltpu.sync_copy(dst_hbm.at[pl.ds(base, IB)], dst_ib)
            for b in range(NBUF):
                pltpu.async_copy(t_hbm.at[src_ib.at[b]], bufs.at[b],
                                 gsems.at[b])
            for j in range(IB):
                b = j % NBUF
                pltpu.make_async_copy(t_hbm.at[src_ib.at[j]], bufs.at[b],
                                      gsems.at[b]).wait()
                pltpu.async_copy(bufs.at[b], acc.at[dst_ib.at[j]],
                                 ssems.at[b], add=True)
                if j + NBUF < IB:
                    pltpu.make_async_copy(bufs.at[b], acc.at[dst_ib.at[j]],
                                          ssems.at[b]).wait()
                    pltpu.async_copy(t_hbm.at[src_ib.at[j + NBUF]],
                                     bufs.at[b], gsems.at[b])
            for j in range(IB - NBUF, IB):
                b = j % NBUF
                pltpu.make_async_copy(bufs.at[b], acc.at[dst_ib.at[j]],
                                      ssems.at[b]).wait()
            return c

        lax.fori_loop(0, nblk, block, 0)
        plsc.subcore_barrier()
        pltpu.sync_copy(acc.at[pl.ds(sid * orows, orows)],
                        out_hbm.at[cid, pl.ds(sid * orows, orows)])

    return pl.kernel(
        body,
        out_type=jax.ShapeDtypeStruct((NC, n_pad, d), jnp.float32),
        mesh=mesh,
        scratch_types=[
            pltpu.VMEM((IB, CHUNK), jnp.int32),
            pltpu.VMEM((IB, CHUNK), jnp.int32),
            pltpu.VMEM((NBUF, CHUNK, d), jnp.float32),
            pltpu.VMEM_SHARED((n_pad, d), jnp.float32),
            pltpu.SemaphoreType.DMA((NBUF,)),
            pltpu.SemaphoreType.DMA((NBUF,)),
        ],
    )


def kernel(x, edge_index, W1, b1, W2, b2):
    n, d = x.shape
    e = edge_index.shape[1]
    d_hid = W1.shape[0]
    d_out = W2.shape[0]

    nw = NC * NS
    rw = _cdiv(_cdiv(_cdiv(e, CHUNK), nw), IB) * IB
    rw0 = rw1 = rw
    e_pad = NS * (rw0 + rw1) * CHUNK
    n_pad = _cdiv(n + 1, 128) * 128

    src = edge_index[0].astype(jnp.int32)
    dst = edge_index[1].astype(jnp.int32)
    pad_src = jnp.arange(e_pad - e, dtype=jnp.int32) % n
    src2d = jnp.concatenate([src, pad_src]).reshape(nw * rw, CHUNK)
    dst2d = jnp.pad(dst, (0, e_pad - e),
                    constant_values=n).reshape(nw * rw, CHUNK)

    n_pad_deg = _cdiv(n + 1, NS * 128) * NS * 128
    degw = _make_sc_deg(n_pad_deg, rw)(dst2d)

    sc_agg = _make_sc_agg(n_pad, d_hid, rw0, rw1)

    blk = _row_blocks(n)
    grid = (_cdiv(n, blk),)
    degw_spec = pl.BlockSpec((NC, blk), lambda i: (0, i))
    row_spec = pl.BlockSpec((blk, d), lambda i: (i, 0))
    w_spec = pl.BlockSpec((d_hid, d), lambda i: (0, 0))
    b_spec = pl.BlockSpec((1, d_hid), lambda i: (0, 0))
    agg_spec = pl.BlockSpec((NC, blk, d_hid), lambda i: (0, i, 0))

    t1_raw = pl.pallas_call(
        _tc_mm_body,
        grid=grid,
        in_specs=[row_spec, w_spec],
        out_specs=pl.BlockSpec((blk, d_hid), lambda i: (i, 0)),
        out_shape=jax.ShapeDtypeStruct((n, d_hid), jnp.float32),
    )(x, W1)
    t1 = pl.pallas_call(
        _tc_scale_body,
        grid=grid,
        in_specs=[pl.BlockSpec((blk, d_hid), lambda i: (i, 0)), degw_spec],
        out_specs=pl.BlockSpec((blk, d_hid), lambda i: (i, 0)),
        out_shape=jax.ShapeDtypeStruct((n, d_hid), jnp.float32),
    )(t1_raw, degw)

    agg1 = sc_agg(t1, src2d, dst2d)

    t2 = pl.pallas_call(
        _tc_mid_body,
        grid=grid,
        in_specs=[agg_spec, degw_spec, b_spec,
                  pl.BlockSpec((d_out, d_hid), lambda i: (0, 0))],
        out_specs=pl.BlockSpec((blk, d_out), lambda i: (i, 0)),
        out_shape=jax.ShapeDtypeStruct((n, d_out), jnp.float32),
    )(agg1, degw, b1.reshape(1, d_hid), W2)

    agg2 = sc_agg(t2, src2d, dst2d)

    out = pl.pallas_call(
        _tc_last_body,
        grid=grid,
        in_specs=[agg_spec, degw_spec, b_spec],
        out_specs=pl.BlockSpec((blk, d_out), lambda i: (i, 0)),
        out_shape=jax.ShapeDtypeStruct((n, d_out), jnp.float32),
    )(agg2, degw, b2.reshape(1, d_out))

    return out


# speedup vs baseline: 1.1423x; 1.0001x over previous
import functools

import jax
import jax.numpy as jnp
import numpy as np
from jax import lax
from jax.experimental import pallas as pl
from jax.experimental.pallas import tpu as pltpu
from jax.experimental.pallas import tpu_sc as plsc

NC = 2
NS = 16
LANE = 16
CHUNK = 128
DEG_W = 128


def _cdiv(a, b):
    return (a + b - 1) // b


def _dinv_col(deg_ref):
    s = deg_ref[0:1, :] + deg_ref[1:2, :]
    dinv = jnp.where(s > 0, lax.rsqrt(jnp.where(s > 0, s, 1.0)), 0.0)
    return lax.transpose(dinv, (1, 0))


def _tc_mm_body(x_ref, w_ref, o_ref):
    o_ref[...] = lax.dot_general(x_ref[...], w_ref[...],
                                 (((1,), (1,)), ((), ())),
                                 preferred_element_type=jnp.float32)


def _tc_scale_body(t_ref, deg_ref, o_ref):
    o_ref[...] = t_ref[...] * _dinv_col(deg_ref)


def _tc_mid_body(agg_ref, deg_ref, b_ref, w_ref, o_ref):
    dinv = _dinv_col(deg_ref)
    h = (agg_ref[0] + agg_ref[1]) * dinv + b_ref[...]
    h = jnp.maximum(h, 0.0)
    t = lax.dot_general(h, w_ref[...], (((1,), (1,)), ((), ())),
                        preferred_element_type=jnp.float32)
    o_ref[...] = t * dinv


def _tc_last_body(agg_ref, deg_ref, b_ref, o_ref):
    dinv = _dinv_col(deg_ref)
    o_ref[...] = (agg_ref[0] + agg_ref[1]) * dinv + b_ref[...]


def _row_blocks(n):
    return min(2048, _cdiv(n, 128) * 128)


def _fill_vmem(ref, rows, w, val):
    v = jnp.full((16,), val, jnp.float32)

    def row(i, c):
        for l in range(w // 16):
            ref[i, pl.ds(l * 16, 16)] = v
        return c

    lax.fori_loop(0, rows, row, 0)


def _zero_acc_slice(zbuf, acc, base, zrows):
    nfull, rem = zrows // CHUNK, zrows % CHUNK
    for k in range(nfull):
        pltpu.sync_copy(zbuf, acc.at[pl.ds(base + k * CHUNK, CHUNK)])
    if rem:
        pltpu.sync_copy(zbuf.at[pl.ds(0, rem)],
                        acc.at[pl.ds(base + nfull * CHUNK, rem)])


def _make_sc_deg(n_pad, rw):
    mesh = plsc.VectorSubcoreMesh(core_axis_name="c", subcore_axis_name="s")
    cols = n_pad // NS

    def body(dst_hbm, out_hbm, dst_all, hist, red, osum, stage):
        cid = lax.axis_index("c")
        sid = lax.axis_index("s")
        wid = cid * NS + sid

        def z(i, c):
            hist[pl.ds(i * 16, 16)] = jnp.zeros((16,), jnp.float32)
            return c

        lax.fori_loop(0, n_pad // 16, z, 0)
        pltpu.sync_copy(dst_hbm.at[pl.ds(wid * rw, rw)], dst_all)
        ones16 = jnp.ones((16,), jnp.float32)

        def chunk(j, c):
            for l in range(CHUNK // 16):
                idx = dst_all[j, pl.ds(l * 16, 16)]
                plsc.addupdate_scatter(hist, [idx], ones16)
            return c

        lax.fori_loop(0, rw, chunk, 0)
        pltpu.sync_copy(hist, stage.at[sid])
        plsc.subcore_barrier()
        pltpu.sync_copy(stage.at[:, pl.ds(sid * cols, cols)], red)

        def col(i, c):
            s = jnp.zeros((16,), jnp.float32)
            for r in range(NS):
                s = s + red[r, pl.ds(i * 16, 16)]
            osum[pl.ds(i * 16, 16)] = s
            return c

        lax.fori_loop(0, cols // 16, col, 0)
        pltpu.sync_copy(osum, out_hbm.at[cid, pl.ds(sid * cols, cols)])

    return pl.kernel(
        body,
        out_type=jax.ShapeDtypeStruct((NC, n_pad), jnp.float32),
        mesh=mesh,
        compiler_params=pltpu.CompilerParams(needs_layout_passes=False),
        scratch_types=[
            pltpu.VMEM((rw, CHUNK), jnp.int32),
            pltpu.VMEM((n_pad,), jnp.float32),
            pltpu.VMEM((NS, cols), jnp.float32),
            pltpu.VMEM((cols,), jnp.float32),
            pltpu.VMEM_SHARED((NS, n_pad), jnp.float32),
        ],
    )


NBUF = 2
IB = 40


def _make_sc_agg(n_pad, d, rw0, rw1):
    mesh = plsc.VectorSubcoreMesh(core_axis_name="c", subcore_axis_name="s")
    zrows = n_pad // NS
    orows = n_pad // NS

    def body(t_hbm, src_hbm, dst_hbm, out_hbm,
             src_ib, dst_ib, bufs, acc, gsems, ssems):
        cid = lax.axis_index("c")
        sid = lax.axis_index("s")
        _fill_vmem(bufs.at[0], CHUNK, d, 0.0)
        _zero_acc_slice(bufs.at[0], acc, sid * zrows, zrows)
        plsc.subcore_barrier()

        is0 = cid == 0
        base_chunk = jnp.where(is0, sid * rw0, NS * rw0 + sid * rw1)
        nblk = jnp.where(is0, rw0 // IB, rw1 // IB)

        def block(g, c):
            base = base_chunk + g * IB
            pltpu.sync_copy(src_hbm.at[pl.ds(base, IB)], src_ib)
            pltpu.sync_copy(dst_hbm.at[pl.ds(base, IB)], dst_ib)
            for b in range(NBUF):
                pltpu.async_copy(t_hbm.at[src_ib.at[b]], bufs.at[b],
                                 gsems.at[b])
            for j in range(IB):
                b = j % NBUF
                pltpu.make_async_copy(t_hbm.at[src_ib.at[j]], bufs.at[b],
                                      gsems.at[b]).wait()
                pltpu.async_copy(bufs.at[b], acc.at[dst_ib.at[j]],
                                 ssems.at[b], add=True)
                if j + NBUF < IB:
                    pltpu.make_async_copy(bufs.at[b], acc.at[dst_ib.at[j]],
                                          ssems.at[b]).wait()
                    pltpu.async_copy(t_hbm.at[src_ib.at[j + NBUF]],
                                     bufs.at[b], gsems.at[b])
            for j in range(IB - NBUF, IB):
                b = j % NBUF
                pltpu.make_async_copy(bufs.at[b], acc.at[dst_ib.at[j]],
                                      ssems.at[b]).wait()
            return c

        lax.fori_loop(0, nblk, block, 0)
        plsc.subcore_barrier()
        pltpu.sync_copy(acc.at[pl.ds(sid * orows, orows)],
                        out_hbm.at[cid, pl.ds(sid * orows, orows)])

    return pl.kernel(
        body,
        out_type=jax.ShapeDtypeStruct((NC, n_pad, d), jnp.float32),
        mesh=mesh,
        scratch_types=[
            pltpu.VMEM((IB, CHUNK), jnp.int32),
            pltpu.VMEM((IB, CHUNK), jnp.int32),
            pltpu.VMEM((NBUF, CHUNK, d), jnp.float32),
            pltpu.VMEM_SHARED((n_pad, d), jnp.float32),
            pltpu.SemaphoreType.DMA((NBUF,)),
            pltpu.SemaphoreType.DMA((NBUF,)),
        ],
    )


def kernel(x, edge_index, W1, b1, W2, b2):
    n, d = x.shape
    e = edge_index.shape[1]
    d_hid = W1.shape[0]
    d_out = W2.shape[0]

    nw = NC * NS
    rw = _cdiv(_cdiv(_cdiv(e, CHUNK), nw), IB) * IB
    rw0 = rw1 = rw
    e_pad = NS * (rw0 + rw1) * CHUNK
    n_pad = _cdiv(n + 1, 128) * 128

    src = edge_index[0].astype(jnp.int32)
    dst = edge_index[1].astype(jnp.int32)
    pad_src = np.arange(e_pad - e, dtype=np.int32) % n
    src2d = jnp.concatenate([src, pad_src]).reshape(nw * rw, CHUNK)
    dst2d = jnp.pad(dst, (0, e_pad - e),
                    constant_values=n).reshape(nw * rw, CHUNK)

    n_pad_deg = _cdiv(n + 1, NS * 128) * NS * 128
    degw = _make_sc_deg(n_pad_deg, rw)(dst2d)

    sc_agg = _make_sc_agg(n_pad, d_hid, rw0, rw1)

    blk = _row_blocks(n)
    grid = (_cdiv(n, blk),)
    degw_spec = pl.BlockSpec((NC, blk), lambda i: (0, i))
    row_spec = pl.BlockSpec((blk, d), lambda i: (i, 0))
    w_spec = pl.BlockSpec((d_hid, d), lambda i: (0, 0))
    b_spec = pl.BlockSpec((1, d_hid), lambda i: (0, 0))
    agg_spec = pl.BlockSpec((NC, blk, d_hid), lambda i: (0, i, 0))

    t1_raw = pl.pallas_call(
        _tc_mm_body,
        grid=grid,
        in_specs=[row_spec, w_spec],
        out_specs=pl.BlockSpec((blk, d_hid), lambda i: (i, 0)),
        out_shape=jax.ShapeDtypeStruct((n, d_hid), jnp.float32),
    )(x, W1)
    t1 = pl.pallas_call(
        _tc_scale_body,
        grid=grid,
        in_specs=[pl.BlockSpec((blk, d_hid), lambda i: (i, 0)), degw_spec],
        out_specs=pl.BlockSpec((blk, d_hid), lambda i: (i, 0)),
        out_shape=jax.ShapeDtypeStruct((n, d_hid), jnp.float32),
    )(t1_raw, degw)

    agg1 = sc_agg(t1, src2d, dst2d)

    t2 = pl.pallas_call(
        _tc_mid_body,
        grid=grid,
        in_specs=[agg_spec, degw_spec, b_spec,
                  pl.BlockSpec((d_out, d_hid), lambda i: (0, 0))],
        out_specs=pl.BlockSpec((blk, d_out), lambda i: (i, 0)),
        out_shape=jax.ShapeDtypeStruct((n, d_out), jnp.float32),
    )(agg1, degw, b1.reshape(1, d_hid), W2)

    agg2 = sc_agg(t2, src2d, dst2d)

    out = pl.pallas_call(
        _tc_last_body,
        grid=grid,
        in_specs=[agg_spec, degw_spec, b_spec],
        out_specs=pl.BlockSpec((blk, d_out), lambda i: (i, 0)),
        out_shape=jax.ShapeDtypeStruct((n, d_out), jnp.float32),
    )(agg2, degw, b2.reshape(1, d_out))

    return out
